# baseline (speedup 1.0000x reference)
import functools

import jax
import jax.numpy as jnp
from jax import lax
from jax.experimental import pallas as pl
from jax.experimental.pallas import tpu as pltpu
from jax.experimental.pallas import tpu_sc as plsc

NC, NS = 2, 16
NW = NC * NS
B = 4 * 4096
D = 1024
BPW = B // NW
CHUNK = 32
NCHUNK = BPW // CHUNK
NBUF = 3

_mesh = plsc.VectorSubcoreMesh(
    core_axis_name="c", subcore_axis_name="s", num_cores=NC, num_subcores=NS
)


@functools.partial(
    pl.kernel,
    out_type=jax.ShapeDtypeStruct((B, D), jnp.float32),
    mesh=_mesh,
    scratch_types=[
        pltpu.VMEM((NCHUNK, CHUNK), jnp.int32),
        pltpu.VMEM((NBUF, CHUNK, D), jnp.float32),
        pltpu.SemaphoreType.DMA,
        pltpu.SemaphoreType.DMA,
        pltpu.SemaphoreType.DMA,
        pltpu.SemaphoreType.DMA,
        pltpu.SemaphoreType.DMA,
        pltpu.SemaphoreType.DMA,
    ],
)
def _embed(tokens_hbm, table_hbm, out_hbm, idx_v, rows_v, g0, g1, g2, s0, s1, s2):
    gsems = (g0, g1, g2)
    ssems = (s0, s1, s2)
    wid = lax.axis_index("s") * NC + lax.axis_index("c")
    base = wid * BPW
    pltpu.sync_copy(tokens_hbm.at[pl.ds(wid * NCHUNK, NCHUNK)], idx_v)
    gds = [
        pltpu.async_copy(table_hbm.at[idx_v.at[j]], rows_v.at[j], gsems[j])
        for j in range(NBUF)
    ]
    sds = [None] * NBUF
    for c in range(NCHUNK):
        b = c % NBUF
        gds[b].wait()
        sds[b] = pltpu.async_copy(
            rows_v.at[b], out_hbm.at[pl.ds(base + c * CHUNK, CHUNK)], ssems[b]
        )
        pc = c - 1
        if pc >= 0 and pc + NBUF < NCHUNK:
            pb = pc % NBUF
            sds[pb].wait()
            gds[pb] = pltpu.async_copy(
                table_hbm.at[idx_v.at[pc + NBUF]], rows_v.at[pb], gsems[pb]
            )
    for c in range(NCHUNK - NBUF, NCHUNK):
        sds[c % NBUF].wait()


def kernel(tokens, W_E):
    flat = tokens.reshape(NW * NCHUNK, CHUNK)
    out = _embed(flat, W_E)
    return out.reshape(tokens.shape + (W_E.shape[1],))


# speedup vs baseline: 1.0194x; 1.0194x over previous
import functools

import jax
import jax.numpy as jnp
from jax import lax
from jax.experimental import pallas as pl
from jax.experimental.pallas import tpu as pltpu
from jax.experimental.pallas import tpu_sc as plsc

NC, NS = 2, 16
NW = NC * NS
S, T = 4, 4096
D = 1024
BPW = (S * T) // NW
WPS = T // BPW
CHUNK = 32
NCHUNK = BPW // CHUNK
NBUF = 3

_mesh = plsc.VectorSubcoreMesh(
    core_axis_name="c", subcore_axis_name="s", num_cores=NC, num_subcores=NS
)


@functools.partial(
    pl.kernel,
    out_type=jax.ShapeDtypeStruct((S, T, D), jnp.float32),
    mesh=_mesh,
    scratch_types=[
        pltpu.VMEM((NCHUNK, CHUNK), jnp.int32),
        pltpu.VMEM((NBUF, CHUNK, D), jnp.float32),
        pltpu.SemaphoreType.DMA,
        pltpu.SemaphoreType.DMA,
        pltpu.SemaphoreType.DMA,
        pltpu.SemaphoreType.DMA,
        pltpu.SemaphoreType.DMA,
        pltpu.SemaphoreType.DMA,
        pltpu.SemaphoreType.DMA,
    ],
)
def _embed(tokens_hbm, table_hbm, out_hbm, idx_v, rows_v, isem, g0, g1, g2, s0, s1, s2):
    gsems = (g0, g1, g2)
    ssems = (s0, s1, s2)
    wid = lax.axis_index("s") * NC + lax.axis_index("c")
    seq = wid // WPS
    col0 = (wid % WPS) * BPW
    ids = [
        pltpu.async_copy(
            tokens_hbm.at[seq, pl.ds(col0 + c * CHUNK, CHUNK)], idx_v.at[c], isem
        )
        for c in range(NCHUNK)
    ]
    for d in ids:
        d.wait()
    gds = [
        pltpu.async_copy(table_hbm.at[idx_v.at[j]], rows_v.at[j], gsems[j])
        for j in range(NBUF)
    ]
    sds = [None] * NBUF
    for c in range(NCHUNK):
        b = c % NBUF
        gds[b].wait()
        sds[b] = pltpu.async_copy(
            rows_v.at[b], out_hbm.at[seq, pl.ds(col0 + c * CHUNK, CHUNK)], ssems[b]
        )
        nc = c + NBUF
        if nc < NCHUNK:
            sds[b].wait()
            gds[b] = pltpu.async_copy(
                table_hbm.at[idx_v.at[nc]], rows_v.at[b], gsems[b]
            )
    for c in range(NCHUNK - NBUF, NCHUNK):
        sds[c % NBUF].wait()


def kernel(tokens, W_E):
    return _embed(tokens, W_E)
